# baseline (speedup 1.0000x reference)
import jax
import jax.numpy as jnp
import numpy as np
from jax import lax
from jax.experimental import pallas as pl
from jax.experimental.pallas import tpu as pltpu
from jax.experimental.pallas import tpu_sc as plsc

B = 4
H = W = 192
C = 96
IN_PER_B = H * W * C
OUT_PER_B = 4 * IN_PER_B
TOTAL_IN = B * IN_PER_B
TOTAL_OUT = B * OUT_PER_B

NC = 2
NS = 16

CHUNKS_PER_B = 8
CHUNK = OUT_PER_B // CHUNKS_PER_B
ROUNDS_PER_B = CHUNKS_PER_B // NC
N_ROUNDS = B * ROUNDS_PER_B

ACC_SIZE = CHUNK
ZSLICE = ACC_SIZE // NS

SLICE = IN_PER_B // NS
WIN = 3072
N_WIN = SLICE // WIN
N_PAIR = N_WIN // 2
FLUSH = CHUNK // NS

_F_THIRD = np.float32(1.0 / 3.0)
_F_BIAS = np.float32(0.1)


def _body(upd_hbm, mask_hbm, zero_hbm, out_hbm,
          mask_a, mask_b, val_a, val_b, idx_a, idx_b, acc,
          sem_ev, sem_od, sem_sc):
    mask_bufs = (mask_a, mask_b)
    val_bufs = (val_a, val_b)
    idx_bufs = (idx_a, idx_b)
    core = lax.axis_index("c")
    s = lax.axis_index("s")
    iota16 = lax.iota(jnp.int32, 16)

    pltpu.sync_copy(zero_hbm.at[pl.ds(s * ZSLICE, ZSLICE)],
                    acc.at[pl.ds(s * ZSLICE, ZSLICE)])
    plsc.subcore_barrier()

    def round_body(rnd, _):
        b = rnd // ROUNDS_PER_B
        r = rnd - b * ROUNDS_PER_B
        chunk_base = (2 * r + core) * CHUNK
        cbv = [iota16 + (jnp.int32(cu) - chunk_base)
               for cu in range(0, 96, 16)]

        win0 = b * IN_PER_B + s * SLICE

        def in_copies(slot, w, sem):
            return (
                pltpu.make_async_copy(
                    mask_hbm.at[pl.ds(win0 + w * WIN, WIN)],
                    mask_bufs[slot], sem),
                pltpu.make_async_copy(
                    upd_hbm.at[pl.ds(win0 + w * WIN, WIN)],
                    val_bufs[slot], sem),
            )

        def fire_loads(slot, w, sem):
            for d in in_copies(slot, w, sem):
                d.start()

        def wait_loads(slot, w, sem):
            for d in in_copies(slot, w, sem):
                d.wait()

        def fire_scatters(slot):
            pltpu.async_copy(val_bufs[slot],
                             acc.at[idx_bufs[slot]],
                             sem_sc, add=True)

        def drain_scatters(slot):
            pltpu.make_async_copy(val_bufs[slot],
                                  acc.at[idx_bufs[slot]],
                                  sem_sc).wait()

        def compute(slot):
            mb = mask_bufs[slot]
            vb = val_bufs[slot]
            ib = idx_bufs[slot]

            @plsc.parallel_loop(0, WIN, 96)
            def g_body(o0):
                for u in range(6):
                    o = o0 + 16 * u
                    m = mb[pl.ds(o, 16)]
                    t2 = lax.shift_right_arithmetic(m, 5)
                    q = (t2.astype(jnp.float32) * _F_THIRD + _F_BIAS
                         ).astype(jnp.int32)
                    rel = q * 96 + cbv[u]
                    valid = (plsc.bitcast(rel, jnp.uint32)
                             < jnp.uint32(CHUNK))
                    ib[pl.ds(o, 16)] = jnp.where(valid, rel,
                                                 rel & jnp.int32(0xFFFFF))
                    vv = vb[pl.ds(o, 16)]
                    vb[pl.ds(o, 16)] = jnp.where(valid, vv,
                                                 jnp.float32(0.0))

        fire_loads(0, 0, sem_ev)

        def pair_body(p, _):
            w0 = 2 * p
            w1 = w0 + 1

            @pl.when(p > 0)
            def _():
                drain_scatters(1)
            fire_loads(1, w1, sem_od)
            wait_loads(0, w0, sem_ev)
            compute(0)
            fire_scatters(0)

            wait_loads(1, w1, sem_od)
            drain_scatters(0)

            @pl.when(p < N_PAIR - 1)
            def _():
                fire_loads(0, w0 + 2, sem_ev)
            compute(1)
            fire_scatters(1)
            return 0

        lax.fori_loop(0, N_PAIR, pair_body, 0)
        drain_scatters(1)
        plsc.subcore_barrier()

        out_base = b * OUT_PER_B + chunk_base + s * FLUSH
        pltpu.sync_copy(acc.at[pl.ds(s * FLUSH, FLUSH)],
                        out_hbm.at[pl.ds(out_base, FLUSH)])
        pltpu.sync_copy(zero_hbm.at[pl.ds(s * ZSLICE, ZSLICE)],
                        acc.at[pl.ds(s * ZSLICE, ZSLICE)])
        plsc.subcore_barrier()
        return 0

    lax.fori_loop(0, N_ROUNDS, round_body, 0)


def kernel(updates, mask):
    updates, mask = lax.optimization_barrier((updates, mask))
    upd_flat = updates.reshape(TOTAL_IN) * np.float32(1.0)
    mask_flat = mask.astype(jnp.int32).reshape(TOTAL_IN) | np.int32(0)
    upd_flat, mask_flat = lax.optimization_barrier((upd_flat, mask_flat))
    zeros = jnp.zeros((ACC_SIZE,), jnp.float32)
    mesh = plsc.VectorSubcoreMesh(core_axis_name="c", subcore_axis_name="s")
    out = pl.kernel(
        _body,
        out_type=jax.ShapeDtypeStruct((TOTAL_OUT,), jnp.float32),
        mesh=mesh,
        scratch_types=[
            pltpu.VMEM((WIN,), jnp.int32),
            pltpu.VMEM((WIN,), jnp.int32),
            pltpu.VMEM((WIN,), jnp.float32),
            pltpu.VMEM((WIN,), jnp.float32),
            pltpu.VMEM((WIN,), jnp.int32),
            pltpu.VMEM((WIN,), jnp.int32),
            pltpu.VMEM_SHARED((ACC_SIZE,), jnp.float32),
            pltpu.SemaphoreType.DMA,
            pltpu.SemaphoreType.DMA,
            pltpu.SemaphoreType.DMA,
        ],
    )(upd_flat, mask_flat, zeros)
    return out.reshape(B, 2 * H, 2 * W, C)
